# Initial kernel scaffold; baseline (speedup 1.0000x reference)
import jax
import jax.numpy as jnp
from jax.experimental import pallas as pl


def kernel(x):
    raise NotImplementedError("write your pallas kernel here")


# speedup vs baseline: 2.4711x; 2.4711x over previous
import functools

import jax
import jax.numpy as jnp
from jax.experimental import pallas as pl
from jax.experimental.pallas import tpu as pltpu


def _rcumsum_block(x, idx, W):
    rc = x
    s = 1
    while s < W:
        rolled = pltpu.roll(rc, W - s, axis=1)
        rc = rc + jnp.where(idx < W - s, rolled, 0.0)
        s *= 2
    return rc


def _kernel(x_ref, o_ref, carry_ref, *, W):
    j = pl.program_id(1)

    @pl.when(j == 0)
    def _():
        carry_ref[...] = jnp.zeros_like(carry_ref)

    x = x_ref[...]
    BR = x.shape[0]
    idx = jax.lax.broadcasted_iota(jnp.int32, (BR, W), 1)
    rc = _rcumsum_block(x, idx, W)
    o_ref[...] = rc + carry_ref[...]
    carry_ref[...] = carry_ref[...] + rc[:, 0:1]


def kernel(x):
    M, N = x.shape
    BR, W = 256, 512
    ncb = N // W
    grid = (M // BR, ncb)
    return pl.pallas_call(
        functools.partial(_kernel, W=W),
        grid=grid,
        in_specs=[pl.BlockSpec((BR, W), lambda i, j: (i, ncb - 1 - j))],
        out_specs=pl.BlockSpec((BR, W), lambda i, j: (i, ncb - 1 - j)),
        out_shape=jax.ShapeDtypeStruct((M, N), x.dtype),
        scratch_shapes=[pltpu.VMEM((BR, 1), jnp.float32)],
    )(x)
